# baseline (speedup 1.0000x reference)
import functools

import jax
import jax.numpy as jnp
from jax import lax
from jax.experimental import pallas as pl
from jax.experimental.pallas import tpu as pltpu
from jax.experimental.pallas import tpu_sc as plsc

N = 10000
E = 160000
F_IN = 256
H = 64
C = 10
G = 64

NC = 2
NS = 16
NW = NC * NS
CHUNK = 128
CH0 = 24
CH1 = 56
NCHT = NS * (CH0 + CH1)
SMAX = max(CH0, CH1)
SROWS = NCHT + SMAX
E_PAD = NCHT * CHUNK
N_DUMP = N
N_ACC = 10240
RPT = N_ACC // NS
DW = 16

_EPS = 1e-5


def _bn(h, g, be):
    m = jnp.mean(h, axis=0, keepdims=True)
    v = jnp.mean((h - m) ** 2, axis=0, keepdims=True)
    return g * (h - m) * lax.rsqrt(v + _EPS) + be


def _mm(a, b):
    return jnp.dot(a, b, preferred_element_type=jnp.float32)


def _tc_in_body(x_ref, w_ref, b_ref, g_ref, be_ref, o_ref):
    h0 = jnp.maximum(_mm(x_ref[...], w_ref[...]) + b_ref[...], 0.0)
    o_ref[...] = _bn(h0, g_ref[...], be_ref[...])


def _combine(h_ref, acc_ref, deg_ref, wb_ref, bb_ref, gb_ref, beb_ref):
    h = h_ref[...]
    s = acc_ref[0, :N, :] + acc_ref[1, :N, :]
    degc = deg_ref[0, :N, 0:1] + deg_ref[1, :N, 0:1]
    pre = _mm(degc * h, wb_ref[0:H, :]) + _mm(s, wb_ref[H:, :]) + bb_ref[...]
    return _bn(jnp.maximum(pre, 0.0), gb_ref[...], beb_ref[...])


def _tc_mid_body(h_ref, acc_ref, deg_ref, wb_ref, bb_ref, gb_ref, beb_ref,
                 wn_ref, bn_ref, gn_ref, ben_ref, o_ref):
    out1 = _combine(h_ref, acc_ref, deg_ref, wb_ref, bb_ref, gb_ref, beb_ref)
    r = jnp.maximum(out1, 0.0)
    h0 = jnp.maximum(_mm(r, wn_ref[...]) + bn_ref[...], 0.0)
    o_ref[...] = _bn(h0, gn_ref[...], ben_ref[...])


def _tc_tail_body(h_ref, acc_ref, deg_ref, wb_ref, bb_ref, gb_ref, beb_ref,
                  batch_ref, wc1_ref, bc1_ref, gc_ref, bec_ref,
                  wc2_ref, bc2_ref, o_ref):
    hf = _combine(h_ref, acc_ref, deg_ref, wb_ref, bb_ref, gb_ref, beb_ref)
    bt = batch_ref[0:1, :]
    gid = lax.broadcasted_iota(jnp.int32, (G, N), 0)
    onehot = (gid == bt).astype(jnp.float32)
    sums = _mm(onehot, hf)
    counts = jnp.sum(onehot, axis=1, keepdims=True)
    pooled = sums / jnp.maximum(counts, 1.0)
    z = _bn(jnp.maximum(_mm(pooled, wc1_ref[...]) + bc1_ref[...], 0.0),
            gc_ref[...], bec_ref[...])
    logits = _mm(z, wc2_ref[...]) + bc2_ref[...]
    mx = jnp.max(logits, axis=1, keepdims=True)
    lse = jnp.log(jnp.sum(jnp.exp(logits - mx), axis=1, keepdims=True)) + mx
    o_ref[...] = logits - lse


_tc_in = pl.pallas_call(
    _tc_in_body, out_shape=jax.ShapeDtypeStruct((N, H), jnp.float32))
_tc_mid = pl.pallas_call(
    _tc_mid_body, out_shape=jax.ShapeDtypeStruct((N, H), jnp.float32))
_tc_tail = pl.pallas_call(
    _tc_tail_body, out_shape=jax.ShapeDtypeStruct((G, C), jnp.float32))


def _sc_spmm_body(with_deg, *refs):
    if with_deg:
        (h_hbm, src_hbm, dst_hbm, z2_hbm, zd_hbm, ones_hbm,
         acc_out, deg_out, idx_s, idx_d, r_a, r_b, r_c, r_d, acc_sh,
         ga, gb_, gc_, gd, sa, sb, sc_, sd, ones_v, deg_sh) = refs
    else:
        (h_hbm, src_hbm, dst_hbm, z2_hbm,
         acc_out, idx_s, idx_d, r_a, r_b, r_c, r_d, acc_sh,
         ga, gb_, gc_, gd, sa, sb, sc_, sd) = refs
    rows = [r_a, r_b, r_c, r_d]
    gsem = [ga, gb_, gc_, gd]
    ssem = [sa, sb, sc_, sd]

    c = lax.axis_index("c")
    s = lax.axis_index("s")
    r0 = pl.multiple_of(s * RPT, 8)
    nch = jnp.where(c == 0, CH0, CH1)
    q0 = jnp.where(c == 0, s * CH0, NS * CH0 + s * CH1)

    pltpu.sync_copy(src_hbm.at[pl.ds(q0, SMAX), :], idx_s)
    pltpu.sync_copy(dst_hbm.at[pl.ds(q0, SMAX), :], idx_d)
    pltpu.sync_copy(z2_hbm.at[pl.ds(r0, RPT), :], acc_sh.at[pl.ds(r0, RPT), :])
    if with_deg:
        pltpu.sync_copy(zd_hbm.at[pl.ds(r0, RPT), :],
                        deg_sh.at[pl.ds(r0, RPT), :])
        pltpu.sync_copy(ones_hbm, ones_v)
    plsc.subcore_barrier()

    def gather(i, b):
        pltpu.async_copy(h_hbm.at[idx_s.at[i]], rows[b], gsem[b])

    def gather_wait(i, b):
        pltpu.make_async_copy(h_hbm.at[idx_s.at[i]], rows[b], gsem[b]).wait()

    def scatter(i, b):
        pltpu.async_copy(rows[b], acc_sh.at[idx_d.at[i]], ssem[b], add=True)
        if with_deg:
            pltpu.sync_copy(ones_v, deg_sh.at[idx_d.at[i]], add=True)

    def scatter_wait(i, b):
        pltpu.make_async_copy(rows[b], acc_sh.at[idx_d.at[i]], ssem[b]).wait()

    gather(0, 0)
    gather(1, 1)

    def quad(j, carry):
        for b in range(4):
            i = 4 * j + b
            gather_wait(i, b)
            scatter(i, b)
            b2 = (b + 2) % 4

            @pl.when(i < nch - 2)
            def _(i=i, b2=b2):
                @pl.when(i >= 2)
                def _():
                    scatter_wait(i - 2, b2)
                gather(i + 2, b2)
        return carry

    lax.fori_loop(0, nch // 4, quad, 0)
    for b in range(4):
        scatter_wait(nch - 4 + b, b)
    plsc.subcore_barrier()

    pltpu.sync_copy(acc_sh.at[pl.ds(r0, RPT), :],
                    acc_out.at[c, pl.ds(r0, RPT), :])
    if with_deg:
        pltpu.sync_copy(deg_sh.at[pl.ds(r0, RPT), :],
                        deg_out.at[c, pl.ds(r0, RPT), :])


_SC_MESH = plsc.VectorSubcoreMesh(core_axis_name="c", subcore_axis_name="s")
_SC_PARAMS = pltpu.CompilerParams(use_tc_tiling_on_sc=False)

_sc_spmm_deg = pl.kernel(
    functools.partial(_sc_spmm_body, True),
    out_type=(jax.ShapeDtypeStruct((NC, N_ACC, H), jnp.float32),
              jax.ShapeDtypeStruct((NC, N_ACC, DW), jnp.float32)),
    mesh=_SC_MESH,
    compiler_params=_SC_PARAMS,
    scratch_types=(
        [pltpu.VMEM((SMAX, CHUNK), jnp.int32)] * 2
        + [pltpu.VMEM((CHUNK, H), jnp.float32)] * 4
        + [pltpu.VMEM_SHARED((N_ACC, H), jnp.float32)]
        + [pltpu.SemaphoreType.DMA] * 8
        + [pltpu.VMEM((CHUNK, DW), jnp.float32),
           pltpu.VMEM_SHARED((N_ACC, DW), jnp.float32)]
    ),
)

_sc_spmm = pl.kernel(
    functools.partial(_sc_spmm_body, False),
    out_type=jax.ShapeDtypeStruct((NC, N_ACC, H), jnp.float32),
    mesh=_SC_MESH,
    compiler_params=_SC_PARAMS,
    scratch_types=(
        [pltpu.VMEM((SMAX, CHUNK), jnp.int32)] * 2
        + [pltpu.VMEM((CHUNK, H), jnp.float32)] * 4
        + [pltpu.VMEM_SHARED((N_ACC, H), jnp.float32)]
        + [pltpu.SemaphoreType.DMA] * 8
    ),
)


def kernel(x, edge_index, batch,
           W1a, b1a, g1a, be1a, W1b, b1b, g1b, be1b,
           W2a, b2a, g2a, be2a, W2b, b2b, g2b, be2b,
           W3a, b3a, g3a, be3a, W3b, b3b, g3b, be3b,
           Wc1, bc1, gc, bec, Wc2, bc2):
    pad = SROWS * CHUNK - E
    srcp = jnp.concatenate([edge_index[0], jnp.zeros((pad,), jnp.int32)])
    srcp = srcp.reshape(SROWS, CHUNK)
    dstp = jnp.concatenate([edge_index[1],
                            jnp.full((pad,), N_DUMP, jnp.int32)])
    dstp = dstp.reshape(SROWS, CHUNK)
    z2 = jnp.zeros((N_ACC, H), jnp.float32)
    zd = jnp.zeros((N_ACC, DW), jnp.float32)
    onesc = jnp.ones((CHUNK, DW), jnp.float32)
    batch2 = jnp.broadcast_to(batch, (8, N))

    def row(v):
        return v.reshape(1, -1)

    h1 = _tc_in(x, W1a, row(b1a), row(g1a), row(be1a))
    acc1, deg = _sc_spmm_deg(h1, srcp, dstp, z2, zd, onesc)
    h2 = _tc_mid(h1, acc1, deg, W1b, row(b1b), row(g1b), row(be1b),
                 W2a, row(b2a), row(g2a), row(be2a))
    acc2 = _sc_spmm(h2, srcp, dstp, z2)
    h3 = _tc_mid(h2, acc2, deg, W2b, row(b2b), row(g2b), row(be2b),
                 W3a, row(b3a), row(g3a), row(be3a))
    acc3 = _sc_spmm(h3, srcp, dstp, z2)
    return _tc_tail(h3, acc3, deg, W3b, row(b3b), row(g3b), row(be3b),
                    batch2, Wc1, row(bc1), row(gc), row(bec),
                    Wc2, row(bc2))


# speedup vs baseline: 1.0531x; 1.0531x over previous
import functools

import jax
import jax.numpy as jnp
from jax import lax
from jax.experimental import pallas as pl
from jax.experimental.pallas import tpu as pltpu
from jax.experimental.pallas import tpu_sc as plsc

N = 10000
E = 160000
F_IN = 256
H = 64
C = 10
G = 64

NC = 2
NS = 16
NW = NC * NS
CHUNK = 128
CH0 = 52
CH1 = 28
NCHT = NS * (CH0 + CH1)
SMAX = max(CH0, CH1)
SROWS = NCHT + SMAX
E_PAD = NCHT * CHUNK
N_DUMP = N
N_ACC = 10240
RPT = N_ACC // NS
DW = 16

_EPS = 1e-5


def _bn(h, g, be):
    m = jnp.mean(h, axis=0, keepdims=True)
    v = jnp.mean((h - m) ** 2, axis=0, keepdims=True)
    return g * (h - m) * lax.rsqrt(v + _EPS) + be


def _mm(a, b):
    return jnp.dot(a, b, preferred_element_type=jnp.float32)


def _tc_in_body(x_ref, w_ref, b_ref, g_ref, be_ref, o_ref):
    h0 = jnp.maximum(_mm(x_ref[...], w_ref[...]) + b_ref[...], 0.0)
    o_ref[...] = _bn(h0, g_ref[...], be_ref[...])


def _combine(h_ref, acc_ref, deg_ref, wb_ref, bb_ref, gb_ref, beb_ref):
    h = h_ref[...]
    s = acc_ref[0, :N, :] + acc_ref[1, :N, :]
    degc = deg_ref[0, :N, 0:1] + deg_ref[1, :N, 0:1]
    pre = _mm(degc * h, wb_ref[0:H, :]) + _mm(s, wb_ref[H:, :]) + bb_ref[...]
    return _bn(jnp.maximum(pre, 0.0), gb_ref[...], beb_ref[...])


def _tc_mid_body(h_ref, acc_ref, deg_ref, wb_ref, bb_ref, gb_ref, beb_ref,
                 wn_ref, bn_ref, gn_ref, ben_ref, o_ref):
    out1 = _combine(h_ref, acc_ref, deg_ref, wb_ref, bb_ref, gb_ref, beb_ref)
    r = jnp.maximum(out1, 0.0)
    h0 = jnp.maximum(_mm(r, wn_ref[...]) + bn_ref[...], 0.0)
    o_ref[...] = _bn(h0, gn_ref[...], ben_ref[...])


def _tc_tail_body(h_ref, acc_ref, deg_ref, wb_ref, bb_ref, gb_ref, beb_ref,
                  batch_ref, wc1_ref, bc1_ref, gc_ref, bec_ref,
                  wc2_ref, bc2_ref, o_ref):
    hf = _combine(h_ref, acc_ref, deg_ref, wb_ref, bb_ref, gb_ref, beb_ref)
    bt = batch_ref[0:1, :]
    gid = lax.broadcasted_iota(jnp.int32, (G, N), 0)
    onehot = (gid == bt).astype(jnp.float32)
    sums = _mm(onehot, hf)
    counts = jnp.sum(onehot, axis=1, keepdims=True)
    pooled = sums / jnp.maximum(counts, 1.0)
    z = _bn(jnp.maximum(_mm(pooled, wc1_ref[...]) + bc1_ref[...], 0.0),
            gc_ref[...], bec_ref[...])
    logits = _mm(z, wc2_ref[...]) + bc2_ref[...]
    mx = jnp.max(logits, axis=1, keepdims=True)
    lse = jnp.log(jnp.sum(jnp.exp(logits - mx), axis=1, keepdims=True)) + mx
    o_ref[...] = logits - lse


_tc_in = pl.pallas_call(
    _tc_in_body, out_shape=jax.ShapeDtypeStruct((N, H), jnp.float32))
_tc_mid = pl.pallas_call(
    _tc_mid_body, out_shape=jax.ShapeDtypeStruct((N, H), jnp.float32))
_tc_tail = pl.pallas_call(
    _tc_tail_body, out_shape=jax.ShapeDtypeStruct((G, C), jnp.float32))


def _sc_spmm_body(with_deg, *refs):
    if with_deg:
        (h_hbm, src_hbm, dst_hbm, z2_hbm, zd_hbm, ones_hbm,
         acc_out, deg_out, idx_s, idx_d, r_a, r_b, r_c, r_d, acc_sh,
         ga, gb_, gc_, gd, sa, sb, sc_, sd, ones_v, deg_sh) = refs
    else:
        (h_hbm, src_hbm, dst_hbm, z2_hbm,
         acc_out, idx_s, idx_d, r_a, r_b, r_c, r_d, acc_sh,
         ga, gb_, gc_, gd, sa, sb, sc_, sd) = refs
    rows = [r_a, r_b, r_c, r_d]
    gsem = [ga, gb_, gc_, gd]
    ssem = [sa, sb, sc_, sd]

    c = lax.axis_index("c")
    s = lax.axis_index("s")
    r0 = pl.multiple_of(s * RPT, 8)
    nch = jnp.where(c == 0, CH0, CH1)
    q0 = jnp.where(c == 0, s * CH0, NS * CH0 + s * CH1)

    pltpu.sync_copy(src_hbm.at[pl.ds(q0, SMAX), :], idx_s)
    pltpu.sync_copy(dst_hbm.at[pl.ds(q0, SMAX), :], idx_d)
    pltpu.sync_copy(z2_hbm.at[pl.ds(r0, RPT), :], acc_sh.at[pl.ds(r0, RPT), :])
    if with_deg:
        pltpu.sync_copy(zd_hbm.at[pl.ds(r0, RPT), :],
                        deg_sh.at[pl.ds(r0, RPT), :])
        pltpu.sync_copy(ones_hbm, ones_v)
    plsc.subcore_barrier()

    def gather(i, b):
        pltpu.async_copy(h_hbm.at[idx_s.at[i]], rows[b], gsem[b])

    def gather_wait(i, b):
        pltpu.make_async_copy(h_hbm.at[idx_s.at[i]], rows[b], gsem[b]).wait()

    def scatter(i, b):
        pltpu.async_copy(rows[b], acc_sh.at[idx_d.at[i]], ssem[b], add=True)
        if with_deg:
            pltpu.sync_copy(ones_v, deg_sh.at[idx_d.at[i]], add=True)

    def scatter_wait(i, b):
        pltpu.make_async_copy(rows[b], acc_sh.at[idx_d.at[i]], ssem[b]).wait()

    gather(0, 0)
    gather(1, 1)

    def quad(j, carry):
        for b in range(4):
            i = 4 * j + b
            gather_wait(i, b)
            scatter(i, b)
            b2 = (b + 2) % 4

            @pl.when(i < nch - 2)
            def _(i=i, b2=b2):
                @pl.when(i >= 2)
                def _():
                    scatter_wait(i - 2, b2)
                gather(i + 2, b2)
        return carry

    lax.fori_loop(0, nch // 4, quad, 0)
    for b in range(4):
        scatter_wait(nch - 4 + b, b)
    plsc.subcore_barrier()

    pltpu.sync_copy(acc_sh.at[pl.ds(r0, RPT), :],
                    acc_out.at[c, pl.ds(r0, RPT), :])
    if with_deg:
        pltpu.sync_copy(deg_sh.at[pl.ds(r0, RPT), :],
                        deg_out.at[c, pl.ds(r0, RPT), :])


_SC_MESH = plsc.VectorSubcoreMesh(core_axis_name="c", subcore_axis_name="s")
_SC_PARAMS = pltpu.CompilerParams(use_tc_tiling_on_sc=False)

_sc_spmm_deg = pl.kernel(
    functools.partial(_sc_spmm_body, True),
    out_type=(jax.ShapeDtypeStruct((NC, N_ACC, H), jnp.float32),
              jax.ShapeDtypeStruct((NC, N_ACC, DW), jnp.float32)),
    mesh=_SC_MESH,
    compiler_params=_SC_PARAMS,
    scratch_types=(
        [pltpu.VMEM((SMAX, CHUNK), jnp.int32)] * 2
        + [pltpu.VMEM((CHUNK, H), jnp.float32)] * 4
        + [pltpu.VMEM_SHARED((N_ACC, H), jnp.float32)]
        + [pltpu.SemaphoreType.DMA] * 8
        + [pltpu.VMEM((CHUNK, DW), jnp.float32),
           pltpu.VMEM_SHARED((N_ACC, DW), jnp.float32)]
    ),
)

_sc_spmm = pl.kernel(
    functools.partial(_sc_spmm_body, False),
    out_type=jax.ShapeDtypeStruct((NC, N_ACC, H), jnp.float32),
    mesh=_SC_MESH,
    compiler_params=_SC_PARAMS,
    scratch_types=(
        [pltpu.VMEM((SMAX, CHUNK), jnp.int32)] * 2
        + [pltpu.VMEM((CHUNK, H), jnp.float32)] * 4
        + [pltpu.VMEM_SHARED((N_ACC, H), jnp.float32)]
        + [pltpu.SemaphoreType.DMA] * 8
    ),
)


def kernel(x, edge_index, batch,
           W1a, b1a, g1a, be1a, W1b, b1b, g1b, be1b,
           W2a, b2a, g2a, be2a, W2b, b2b, g2b, be2b,
           W3a, b3a, g3a, be3a, W3b, b3b, g3b, be3b,
           Wc1, bc1, gc, bec, Wc2, bc2):
    pad = SROWS * CHUNK - E
    srcp = jnp.concatenate([edge_index[0], jnp.zeros((pad,), jnp.int32)])
    srcp = srcp.reshape(SROWS, CHUNK)
    dstp = jnp.concatenate([edge_index[1],
                            jnp.full((pad,), N_DUMP, jnp.int32)])
    dstp = dstp.reshape(SROWS, CHUNK)
    z2 = jnp.zeros((N_ACC, H), jnp.float32)
    zd = jnp.zeros((N_ACC, DW), jnp.float32)
    onesc = jnp.ones((CHUNK, DW), jnp.float32)
    batch2 = jnp.broadcast_to(batch, (8, N))

    def row(v):
        return v.reshape(1, -1)

    h1 = _tc_in(x, W1a, row(b1a), row(g1a), row(be1a))
    acc1, deg = _sc_spmm_deg(h1, srcp, dstp, z2, zd, onesc)
    h2 = _tc_mid(h1, acc1, deg, W1b, row(b1b), row(g1b), row(be1b),
                 W2a, row(b2a), row(g2a), row(be2a))
    acc2 = _sc_spmm(h2, srcp, dstp, z2)
    h3 = _tc_mid(h2, acc2, deg, W2b, row(b2b), row(g2b), row(be2b),
                 W3a, row(b3a), row(g3a), row(be3a))
    acc3 = _sc_spmm(h3, srcp, dstp, z2)
    return _tc_tail(h3, acc3, deg, W3b, row(b3b), row(g3b), row(be3b),
                    batch2, Wc1, row(bc1), row(gc), row(bec),
                    Wc2, row(bc2))
